# Initial kernel scaffold; baseline (speedup 1.0000x reference)
import jax
import jax.numpy as jnp
from jax.experimental import pallas as pl


def kernel(pos, edge_attr, knn_label, edge_index, W_node, b_node, W_dist, b_dist, W_knn, Wn0, bn0, Wn1, bn1, We0, be0, We1, be1, We2, be2, Wm1, bm1, a_prelu, Wm2, bm2):
    raise NotImplementedError("write your pallas kernel here")


# speedup vs baseline: 1.4248x; 1.4248x over previous
import functools

import jax
import jax.numpy as jnp
from jax import lax
from jax.experimental import pallas as pl
from jax.experimental.pallas import tpu as pltpu
from jax.experimental.pallas import tpu_sc as plsc

N = 50000
E = 800000
U = 64
H = 32
P = 128

NC = 2
NS = 16
NW = NC * NS

EB = 4000
NB = 5000

_mesh = plsc.VectorSubcoreMesh(core_axis_name="c", subcore_axis_name="s")


def _node0_body(pos_ref, w_ref, b_ref, xp_ref):
    p = pos_ref[...]
    w = w_ref[...]
    x = p[:, 0:1] * w[0:1, :] + p[:, 1:2] * w[1:2, :] + b_ref[...]
    xp_ref[...] = jnp.concatenate([x, jnp.zeros_like(x)], axis=1)


def _node0(pos, w, b):
    return pl.pallas_call(
        _node0_body,
        grid=(N // NB,),
        in_specs=[
            pl.BlockSpec((NB, 2), lambda i: (i, 0)),
            pl.BlockSpec((2, U), lambda i: (0, 0)),
            pl.BlockSpec((1, U), lambda i: (0, 0)),
        ],
        out_specs=pl.BlockSpec((NB, P), lambda i: (i, 0)),
        out_shape=jax.ShapeDtypeStruct((N, P), jnp.float32),
    )(pos, w, b)


def _node_upd_body(xp_ref, aggh_ref, w_ref, b_ref, xn_ref):
    x = xp_ref[:, :U]
    agg = jnp.concatenate([aggh_ref[0], aggh_ref[1]], axis=1)
    t = jnp.dot(x + agg, w_ref[...], preferred_element_type=jnp.float32)
    xn = x + jnp.maximum(t + b_ref[...], 0.0)
    xn_ref[...] = jnp.concatenate([xn, jnp.zeros_like(xn)], axis=1)


def _node_upd(xp, aggh, w, b):
    return pl.pallas_call(
        _node_upd_body,
        grid=(N // NB,),
        in_specs=[
            pl.BlockSpec((NB, P), lambda i: (i, 0)),
            pl.BlockSpec((2, NB, H), lambda i: (0, i, 0)),
            pl.BlockSpec((U, U), lambda i: (0, 0)),
            pl.BlockSpec((1, U), lambda i: (0, 0)),
        ],
        out_specs=pl.BlockSpec((NB, P), lambda i: (i, 0)),
        out_shape=jax.ShapeDtypeStruct((N, P), jnp.float32),
    )(xp, aggh, w, b)


def _edge_init_body(ea_ref, knn_ref, xs_ref, wd_ref, bd_ref, wk_ref,
                    eh_ref, mh_ref):
    lo = ea_ref[...] * wd_ref[...] + bd_ref[...]
    hi = jnp.dot(knn_ref[...], wk_ref[...], preferred_element_type=jnp.float32)
    xs = xs_ref[...]
    eh_ref[0] = lo
    eh_ref[1] = hi
    mh_ref[0] = xs[:, :H] * lo
    mh_ref[1] = xs[:, H:] * hi


def _edge_init(ea, knn, xs, wd, bd, wk):
    return pl.pallas_call(
        _edge_init_body,
        grid=(E // EB,),
        in_specs=[
            pl.BlockSpec((EB, 1), lambda i: (i, 0)),
            pl.BlockSpec((EB, 16), lambda i: (i, 0)),
            pl.BlockSpec((EB, U), lambda i: (i, 0)),
            pl.BlockSpec((1, H), lambda i: (0, 0)),
            pl.BlockSpec((1, H), lambda i: (0, 0)),
            pl.BlockSpec((16, H), lambda i: (0, 0)),
        ],
        out_specs=[
            pl.BlockSpec((2, EB, H), lambda i: (0, i, 0)),
            pl.BlockSpec((2, EB, H), lambda i: (0, i, 0)),
        ],
        out_shape=[
            jax.ShapeDtypeStruct((2, E, H), jnp.float32),
            jax.ShapeDtypeStruct((2, E, H), jnp.float32),
        ],
    )(ea, knn, xs, wd, bd, wk)


def _edge_upd_body(eh_ref, sum_ref, xs_ref, w_ref, b_ref, enh_ref, mh_ref):
    e = jnp.concatenate([eh_ref[0], eh_ref[1]], axis=1)
    s = sum_ref[...] + e
    t = jnp.dot(s, w_ref[...], preferred_element_type=jnp.float32)
    en = e + jnp.maximum(t + b_ref[...], 0.0)
    msg = xs_ref[...] * en
    enh_ref[0] = en[:, :H]
    enh_ref[1] = en[:, H:]
    mh_ref[0] = msg[:, :H]
    mh_ref[1] = msg[:, H:]


def _edge_upd(eh, s, xs, w, b):
    return pl.pallas_call(
        _edge_upd_body,
        grid=(E // EB,),
        in_specs=[
            pl.BlockSpec((2, EB, H), lambda i: (0, i, 0)),
            pl.BlockSpec((EB, U), lambda i: (i, 0)),
            pl.BlockSpec((EB, U), lambda i: (i, 0)),
            pl.BlockSpec((U, U), lambda i: (0, 0)),
            pl.BlockSpec((1, U), lambda i: (0, 0)),
        ],
        out_specs=[
            pl.BlockSpec((2, EB, H), lambda i: (0, i, 0)),
            pl.BlockSpec((2, EB, H), lambda i: (0, i, 0)),
        ],
        out_shape=[
            jax.ShapeDtypeStruct((2, E, H), jnp.float32),
            jax.ShapeDtypeStruct((2, E, H), jnp.float32),
        ],
    )(eh, s, xs, w, b)


def _tail_body(eh_ref, sum_ref, w1_ref, b1_ref, w2_ref, b2_ref,
               wm1_ref, bm1_ref, a_ref, wm2_ref, bm2_ref, o_ref):
    e1 = jnp.concatenate([eh_ref[0], eh_ref[1]], axis=1)
    s2 = sum_ref[...]
    t1 = jnp.dot(s2 + e1, w1_ref[...], preferred_element_type=jnp.float32)
    e2 = e1 + jnp.maximum(t1 + b1_ref[...], 0.0)
    t2 = jnp.dot(s2 + e2, w2_ref[...], preferred_element_type=jnp.float32)
    e3 = e2 + jnp.maximum(t2 + b2_ref[...], 0.0)
    h = jnp.dot(e3, wm1_ref[...], preferred_element_type=jnp.float32) + bm1_ref[...]
    a = a_ref[0, 0]
    h = jnp.where(h >= 0, h, a * h)
    o_ref[...] = jnp.dot(h, wm2_ref[...], preferred_element_type=jnp.float32) + bm2_ref[...]


def _tail(eh, s, w1, b1, w2, b2, wm1, bm1, a, wm2, bm2):
    return pl.pallas_call(
        _tail_body,
        grid=(E // EB,),
        in_specs=[
            pl.BlockSpec((2, EB, H), lambda i: (0, i, 0)),
            pl.BlockSpec((EB, U), lambda i: (i, 0)),
            pl.BlockSpec((U, U), lambda i: (0, 0)),
            pl.BlockSpec((1, U), lambda i: (0, 0)),
            pl.BlockSpec((U, U), lambda i: (0, 0)),
            pl.BlockSpec((1, U), lambda i: (0, 0)),
            pl.BlockSpec((U, H), lambda i: (0, 0)),
            pl.BlockSpec((1, H), lambda i: (0, 0)),
            pl.BlockSpec((1, 1), lambda i: (0, 0)),
            pl.BlockSpec((H, 1), lambda i: (0, 0)),
            pl.BlockSpec((1, 1), lambda i: (0, 0)),
        ],
        out_specs=pl.BlockSpec((EB, 1), lambda i: (i, 0)),
        out_shape=jax.ShapeDtypeStruct((E, 1), jnp.float32),
    )(eh, s, w1, b1, w2, b2, wm1, bm1, a, wm2, bm2)


C1 = 128

EW = E // NW
NCH_W = EW // C1
CR_W = EW - NCH_W * C1

ES = E // NS
NCH_S = ES // C1
CR_S = ES - NCH_S * C1

_SEG_ON = True
NH = 25000
SRB = 1664
AROWS = 16 * SRB


def _make_gsum(want_xs, want_sum):
    out_type = []
    if want_xs:
        out_type.append(jax.ShapeDtypeStruct((E, U), jnp.float32))
    if want_sum:
        out_type.append(jax.ShapeDtypeStruct((E, U), jnp.float32))
    n_out = len(out_type)
    if n_out == 1:
        out_type = out_type[0]

    scratch = [
        pltpu.VMEM((C1,), jnp.int32),
        pltpu.VMEM((C1,), jnp.int32),
        pltpu.VMEM((C1, P), jnp.float32),
        pltpu.VMEM((C1, U), jnp.float32),
        pltpu.VMEM((CR_W,), jnp.int32),
        pltpu.VMEM((CR_W,), jnp.int32),
        pltpu.VMEM((CR_W, P), jnp.float32),
        pltpu.VMEM((CR_W, U), jnp.float32),
    ]

    @functools.partial(pl.kernel, out_type=out_type, mesh=_mesh,
                       scratch_types=scratch)
    def gsum(xp_hbm, src_hbm, dst_hbm, *rest):
        outs = list(rest[:n_out])
        si, di, buf, stg, si2, di2, buf2, stg2 = rest[n_out:]
        xs_hbm = outs.pop(0) if want_xs else None
        sum_hbm = outs.pop(0) if want_sum else None

        c = lax.axis_index("c")
        s = lax.axis_index("s")
        wid = s * NC + c
        base0 = wid * EW

        def extract(buf_, stg_, n):
            @pl.loop(0, n)
            def _(r):
                for cc in range(U // 16):
                    sl = pl.ds(cc * 16, 16)
                    stg_[r, sl] = buf_[r, sl]

        def chunk(b, si_, di_, buf_, stg_, n):
            pltpu.sync_copy(src_hbm.at[pl.ds(b, n)], si_)
            pltpu.sync_copy(xp_hbm.at[si_], buf_)
            if want_xs:
                extract(buf_, stg_, n)
                pltpu.sync_copy(stg_, xs_hbm.at[pl.ds(b, n)])
            if want_sum:
                pltpu.sync_copy(dst_hbm.at[pl.ds(b, n)], di_)
                pltpu.sync_copy(xp_hbm.at[di_], buf_, add=True)
                extract(buf_, stg_, n)
                pltpu.sync_copy(stg_, sum_hbm.at[pl.ds(b, n)])

        @pl.loop(0, NCH_W)
        def _(k):
            chunk(base0 + k * C1, si, di, buf, stg, C1)

        chunk(base0 + NCH_W * C1, si2, di2, buf2, stg2, CR_W)

    return gsum


_gsum_x = _make_gsum(True, False)
_gsum_xs = _make_gsum(True, True)
_gsum_s = _make_gsum(False, True)


@functools.partial(
    pl.kernel,
    out_type=jax.ShapeDtypeStruct((2, 2, AROWS, H), jnp.float32),
    mesh=_mesh,
    scratch_types=[
        pltpu.VMEM((C1,), jnp.int32),
        pltpu.VMEM((C1, H), jnp.float32),
        pltpu.VMEM((CR_S,), jnp.int32),
        pltpu.VMEM((CR_S, H), jnp.float32),
        pltpu.VMEM((C1,), jnp.int32),
        pltpu.VMEM((C1, H), jnp.float32),
        pltpu.VMEM((C1,), jnp.int32),
        pltpu.VMEM((CR_S,), jnp.int32),
        pltpu.VMEM_SHARED((AROWS, H), jnp.float32),
    ],
)
def _seg(mh_hbm, dst_hbm, agg_hbm, di, mb, di2, mb2, zi, zb, dr, dr2, acc):
    c = lax.axis_index("c")
    s = lax.axis_index("s")
    iota = lax.iota(jnp.int32, 16)

    for p in (0, 1):
        @pl.loop(0, C1)
        def _(r):
            zb[r, pl.ds(0, 16)] = jnp.zeros((16,), jnp.float32)
            zb[r, pl.ds(16, 16)] = jnp.zeros((16,), jnp.float32)

        @pl.loop(0, SRB // C1)
        def _(j):
            base = s * SRB + j * C1
            for k in range(C1 // 16):
                zi[pl.ds(k * 16, 16)] = base + k * 16 + iota
            pltpu.sync_copy(zb, acc.at[zi])

        plsc.subcore_barrier()

        base0 = s * ES

        def chunk(b, di_, mb_, dr_, n):
            pltpu.sync_copy(dst_hbm.at[pl.ds(b, n)], di_)
            pltpu.sync_copy(mh_hbm.at[pl.ds(c * E + b, n)], mb_)
            @pl.loop(0, n // 16)
            def _(g):
                sl = pl.ds(g * 16, 16)
                v = di_[sl]
                if p == 0:
                    dr_[sl] = jnp.minimum(jnp.maximum(v, 0), NH)
                else:
                    dr_[sl] = jnp.maximum(jnp.minimum(2 * NH - 1 - v, NH), 0)
            pltpu.sync_copy(mb_, acc.at[dr_], add=True)

        if _SEG_ON:
            @pl.loop(0, NCH_S)
            def _(k):
                chunk(base0 + k * C1, di, mb, dr, C1)

            chunk(base0 + NCH_S * C1, di2, mb2, dr2, CR_S)

        plsc.subcore_barrier()

        @pl.loop(0, SRB // C1)
        def _(j):
            base = s * SRB + j * C1
            for k in range(C1 // 16):
                zi[pl.ds(k * 16, 16)] = base + k * 16 + iota
            pltpu.sync_copy(acc.at[zi], zb)
            pltpu.sync_copy(zb, agg_hbm.at[c, p, pl.ds(base, C1)])


def _unseg(agg):
    return jnp.concatenate([agg[:, 0, :NH], agg[:, 1, :NH][:, ::-1]], axis=1)


def kernel(pos, edge_attr, knn_label, edge_index, W_node, b_node, W_dist, b_dist, W_knn,
           Wn0, bn0, Wn1, bn1, We0, be0, We1, be1, We2, be2, Wm1, bm1, a_prelu, Wm2, bm2):
    src = edge_index[0]
    dst = edge_index[1]

    x0p = _node0(pos, W_node, b_node.reshape(1, U))
    _DBG = False
    if _DBG:
        xs0 = jnp.take(x0p[:, :U], src, axis=0)
    else:
        xs0 = _gsum_x(x0p, src, dst)
    e0h, m0h = _edge_init(edge_attr.reshape(E, 1), knn_label, xs0,
                          W_dist, b_dist.reshape(1, H), W_knn)

    agg0h = _unseg(_seg(m0h.reshape(2 * E, H), dst))
    x1p = _node_upd(x0p, agg0h, Wn0, bn0.reshape(1, U))

    if _DBG:
        xs1 = jnp.take(x1p[:, :U], src, axis=0)
        sum1 = xs1 + jnp.take(x1p[:, :U], dst, axis=0)
    else:
        xs1, sum1 = _gsum_xs(x1p, src, dst)
    e1h, m1h = _edge_upd(e0h, sum1, xs1, We0, be0.reshape(1, U))

    agg1h = _unseg(_seg(m1h.reshape(2 * E, H), dst))
    x2p = _node_upd(x1p, agg1h, Wn1, bn1.reshape(1, U))

    if _DBG:
        sum2 = jnp.take(x2p[:, :U], src, axis=0) + jnp.take(x2p[:, :U], dst, axis=0)
    else:
        sum2 = _gsum_s(x2p, src, dst)
    out = _tail(e1h, sum2, We1, be1.reshape(1, U), We2, be2.reshape(1, U),
                Wm1, bm1.reshape(1, H), a_prelu.reshape(1, 1), Wm2, bm2.reshape(1, 1))
    return out


# speedup vs baseline: 1.7905x; 1.2566x over previous
import functools

import jax
import jax.numpy as jnp
from jax import lax
from jax.experimental import pallas as pl
from jax.experimental.pallas import tpu as pltpu
from jax.experimental.pallas import tpu_sc as plsc

N = 50000
E = 800000
U = 64
H = 32
P = 128

NC = 2
NS = 16
NW = NC * NS

EB = 4000
NB = 5000

_mesh = plsc.VectorSubcoreMesh(core_axis_name="c", subcore_axis_name="s")


def _node0_body(pos_ref, w_ref, b_ref, xp_ref):
    p = pos_ref[...]
    w = w_ref[...]
    x = p[:, 0:1] * w[0:1, :] + p[:, 1:2] * w[1:2, :] + b_ref[...]
    xp_ref[...] = jnp.concatenate([x, jnp.zeros_like(x)], axis=1)


def _node0(pos, w, b):
    return pl.pallas_call(
        _node0_body,
        grid=(N // NB,),
        in_specs=[
            pl.BlockSpec((NB, 2), lambda i: (i, 0)),
            pl.BlockSpec((2, U), lambda i: (0, 0)),
            pl.BlockSpec((1, U), lambda i: (0, 0)),
        ],
        out_specs=pl.BlockSpec((NB, P), lambda i: (i, 0)),
        out_shape=jax.ShapeDtypeStruct((N, P), jnp.float32),
    )(pos, w, b)


def _node_upd_body(xp_ref, aggh_ref, w_ref, b_ref, xn_ref):
    x = xp_ref[:, :U]
    agg = jnp.concatenate([aggh_ref[0], aggh_ref[1]], axis=1)
    t = jnp.dot(x + agg, w_ref[...], preferred_element_type=jnp.float32)
    xn = x + jnp.maximum(t + b_ref[...], 0.0)
    xn_ref[...] = jnp.concatenate([xn, jnp.zeros_like(xn)], axis=1)


def _node_upd(xp, aggh, w, b):
    return pl.pallas_call(
        _node_upd_body,
        grid=(N // NB,),
        in_specs=[
            pl.BlockSpec((NB, P), lambda i: (i, 0)),
            pl.BlockSpec((2, NB, H), lambda i: (0, i, 0)),
            pl.BlockSpec((U, U), lambda i: (0, 0)),
            pl.BlockSpec((1, U), lambda i: (0, 0)),
        ],
        out_specs=pl.BlockSpec((NB, P), lambda i: (i, 0)),
        out_shape=jax.ShapeDtypeStruct((N, P), jnp.float32),
    )(xp, aggh, w, b)


def _edge_init_body(ea_ref, knn_ref, xs_ref, wd_ref, bd_ref, wk_ref,
                    eh_ref, mh_ref):
    lo = ea_ref[...] * wd_ref[...] + bd_ref[...]
    hi = jnp.dot(knn_ref[...], wk_ref[...], preferred_element_type=jnp.float32)
    xs = xs_ref[...]
    eh_ref[0] = lo
    eh_ref[1] = hi
    mh_ref[0] = xs[:, :H] * lo
    mh_ref[1] = xs[:, H:] * hi


def _edge_init(ea, knn, xs, wd, bd, wk):
    return pl.pallas_call(
        _edge_init_body,
        grid=(E // EB,),
        in_specs=[
            pl.BlockSpec((EB, 1), lambda i: (i, 0)),
            pl.BlockSpec((EB, 16), lambda i: (i, 0)),
            pl.BlockSpec((EB, U), lambda i: (i, 0)),
            pl.BlockSpec((1, H), lambda i: (0, 0)),
            pl.BlockSpec((1, H), lambda i: (0, 0)),
            pl.BlockSpec((16, H), lambda i: (0, 0)),
        ],
        out_specs=[
            pl.BlockSpec((2, EB, H), lambda i: (0, i, 0)),
            pl.BlockSpec((2, EB, H), lambda i: (0, i, 0)),
        ],
        out_shape=[
            jax.ShapeDtypeStruct((2, E, H), jnp.float32),
            jax.ShapeDtypeStruct((2, E, H), jnp.float32),
        ],
    )(ea, knn, xs, wd, bd, wk)


def _edge_upd_body(eh_ref, sum_ref, xs_ref, w_ref, b_ref, enh_ref, mh_ref):
    e = jnp.concatenate([eh_ref[0], eh_ref[1]], axis=1)
    s = sum_ref[...] + e
    t = jnp.dot(s, w_ref[...], preferred_element_type=jnp.float32)
    en = e + jnp.maximum(t + b_ref[...], 0.0)
    msg = xs_ref[...] * en
    enh_ref[0] = en[:, :H]
    enh_ref[1] = en[:, H:]
    mh_ref[0] = msg[:, :H]
    mh_ref[1] = msg[:, H:]


def _edge_upd(eh, s, xs, w, b):
    return pl.pallas_call(
        _edge_upd_body,
        grid=(E // EB,),
        in_specs=[
            pl.BlockSpec((2, EB, H), lambda i: (0, i, 0)),
            pl.BlockSpec((EB, U), lambda i: (i, 0)),
            pl.BlockSpec((EB, U), lambda i: (i, 0)),
            pl.BlockSpec((U, U), lambda i: (0, 0)),
            pl.BlockSpec((1, U), lambda i: (0, 0)),
        ],
        out_specs=[
            pl.BlockSpec((2, EB, H), lambda i: (0, i, 0)),
            pl.BlockSpec((2, EB, H), lambda i: (0, i, 0)),
        ],
        out_shape=[
            jax.ShapeDtypeStruct((2, E, H), jnp.float32),
            jax.ShapeDtypeStruct((2, E, H), jnp.float32),
        ],
    )(eh, s, xs, w, b)


def _tail_body(eh_ref, sum_ref, w1_ref, b1_ref, w2_ref, b2_ref,
               wm1_ref, bm1_ref, a_ref, wm2_ref, bm2_ref, o_ref):
    e1 = jnp.concatenate([eh_ref[0], eh_ref[1]], axis=1)
    s2 = sum_ref[...]
    t1 = jnp.dot(s2 + e1, w1_ref[...], preferred_element_type=jnp.float32)
    e2 = e1 + jnp.maximum(t1 + b1_ref[...], 0.0)
    t2 = jnp.dot(s2 + e2, w2_ref[...], preferred_element_type=jnp.float32)
    e3 = e2 + jnp.maximum(t2 + b2_ref[...], 0.0)
    h = jnp.dot(e3, wm1_ref[...], preferred_element_type=jnp.float32) + bm1_ref[...]
    a = a_ref[0, 0]
    h = jnp.where(h >= 0, h, a * h)
    o_ref[...] = jnp.dot(h, wm2_ref[...], preferred_element_type=jnp.float32) + bm2_ref[...]


def _tail(eh, s, w1, b1, w2, b2, wm1, bm1, a, wm2, bm2):
    return pl.pallas_call(
        _tail_body,
        grid=(E // EB,),
        in_specs=[
            pl.BlockSpec((2, EB, H), lambda i: (0, i, 0)),
            pl.BlockSpec((EB, U), lambda i: (i, 0)),
            pl.BlockSpec((U, U), lambda i: (0, 0)),
            pl.BlockSpec((1, U), lambda i: (0, 0)),
            pl.BlockSpec((U, U), lambda i: (0, 0)),
            pl.BlockSpec((1, U), lambda i: (0, 0)),
            pl.BlockSpec((U, H), lambda i: (0, 0)),
            pl.BlockSpec((1, H), lambda i: (0, 0)),
            pl.BlockSpec((1, 1), lambda i: (0, 0)),
            pl.BlockSpec((H, 1), lambda i: (0, 0)),
            pl.BlockSpec((1, 1), lambda i: (0, 0)),
        ],
        out_specs=pl.BlockSpec((EB, 1), lambda i: (i, 0)),
        out_shape=jax.ShapeDtypeStruct((E, 1), jnp.float32),
    )(eh, s, w1, b1, w2, b2, wm1, bm1, a, wm2, bm2)


C1 = 128

EW = E // NW
NCH_W = EW // C1
CR_W = EW - NCH_W * C1

ES = E // NS
NCH_S = ES // C1
CR_S = ES - NCH_S * C1

_SEG_ON = True
NH = 25000
SRB = 1664
AROWS = 16 * SRB


def _make_gsum(want_xs, want_sum):
    out_type = []
    if want_xs:
        out_type.append(jax.ShapeDtypeStruct((E, U), jnp.float32))
    if want_sum:
        out_type.append(jax.ShapeDtypeStruct((E, U), jnp.float32))
    n_out = len(out_type)
    if n_out == 1:
        out_type = out_type[0]

    scratch = []
    for _ in range(2):
        scratch += [
            pltpu.VMEM((C1,), jnp.int32),
            pltpu.VMEM((C1,), jnp.int32),
            pltpu.VMEM((C1, P), jnp.float32),
            pltpu.VMEM((C1, P), jnp.float32),
            pltpu.SemaphoreType.DMA,
            pltpu.SemaphoreType.DMA,
        ]
    scratch += [
        pltpu.VMEM((C1, U), jnp.float32),
        pltpu.VMEM((CR_W,), jnp.int32),
        pltpu.VMEM((CR_W,), jnp.int32),
        pltpu.VMEM((CR_W, P), jnp.float32),
        pltpu.VMEM((CR_W, U), jnp.float32),
    ]

    @functools.partial(pl.kernel, out_type=out_type, mesh=_mesh,
                       scratch_types=scratch)
    def gsum(xp_hbm, src_hbm, dst_hbm, *rest):
        outs = list(rest[:n_out])
        (si0, di0, bs0, bd0, ss0, sd0,
         si1, di1, bs1, bd1, ss1, sd1,
         stg, si2, di2, buf2, stg2) = rest[n_out:]
        xs_hbm = outs.pop(0) if want_xs else None
        sum_hbm = outs.pop(0) if want_sum else None
        slots = ((si0, di0, bs0, bd0, ss0, sd0),
                 (si1, di1, bs1, bd1, ss1, sd1))

        c = lax.axis_index("c")
        s = lax.axis_index("s")
        wid = s * NC + c
        base0 = wid * EW

        def issue(k, slot):
            si_, di_, bs_, bd_, ss_, sd_ = slot
            b = base0 + k * C1
            pltpu.sync_copy(src_hbm.at[pl.ds(b, C1)], si_)
            pltpu.async_copy(xp_hbm.at[si_], bs_, ss_)
            if want_sum:
                pltpu.sync_copy(dst_hbm.at[pl.ds(b, C1)], di_)
                pltpu.async_copy(xp_hbm.at[di_], bd_, sd_)

        def finish(k, slot):
            si_, di_, bs_, bd_, ss_, sd_ = slot
            b = base0 + k * C1
            pltpu.make_async_copy(xp_hbm.at[si_], bs_, ss_).wait()
            if want_sum:
                pltpu.make_async_copy(xp_hbm.at[di_], bd_, sd_).wait()
            if want_xs:
                @pl.loop(0, C1)
                def _(r):
                    for cc in range(U // 16):
                        sl = pl.ds(cc * 16, 16)
                        stg[r, sl] = bs_[r, sl]
                pltpu.sync_copy(stg, xs_hbm.at[pl.ds(b, C1)])
            if want_sum:
                @pl.loop(0, C1)
                def _(r):
                    for cc in range(U // 16):
                        sl = pl.ds(cc * 16, 16)
                        stg[r, sl] = bs_[r, sl] + bd_[r, sl]
                pltpu.sync_copy(stg, sum_hbm.at[pl.ds(b, C1)])

        issue(0, slots[0])

        @pl.loop(0, (NCH_W - 1) // 2)
        def _(j):
            k0 = 2 * j
            issue(k0 + 1, slots[1])
            finish(k0, slots[0])
            issue(k0 + 2, slots[0])
            finish(k0 + 1, slots[1])

        finish(NCH_W - 1, slots[0])

        b = base0 + NCH_W * C1
        pltpu.sync_copy(src_hbm.at[pl.ds(b, CR_W)], si2)
        pltpu.sync_copy(xp_hbm.at[si2], buf2)
        if want_xs:
            @pl.loop(0, CR_W)
            def _(r):
                for cc in range(U // 16):
                    sl = pl.ds(cc * 16, 16)
                    stg2[r, sl] = buf2[r, sl]
            pltpu.sync_copy(stg2, xs_hbm.at[pl.ds(b, CR_W)])
        if want_sum:
            pltpu.sync_copy(dst_hbm.at[pl.ds(b, CR_W)], di2)
            pltpu.sync_copy(xp_hbm.at[di2], buf2, add=True)
            @pl.loop(0, CR_W)
            def _(r):
                for cc in range(U // 16):
                    sl = pl.ds(cc * 16, 16)
                    stg2[r, sl] = buf2[r, sl]
            pltpu.sync_copy(stg2, sum_hbm.at[pl.ds(b, CR_W)])

    return gsum


_gsum_x = _make_gsum(True, False)
_gsum_xs = _make_gsum(True, True)
_gsum_s = _make_gsum(False, True)


@functools.partial(
    pl.kernel,
    out_type=jax.ShapeDtypeStruct((2, 2, AROWS, H), jnp.float32),
    mesh=_mesh,
    scratch_types=[
        pltpu.VMEM((C1,), jnp.int32),
        pltpu.VMEM((C1, H), jnp.float32),
        pltpu.VMEM((C1,), jnp.int32),
        pltpu.SemaphoreType.DMA,
        pltpu.VMEM((C1,), jnp.int32),
        pltpu.VMEM((C1, H), jnp.float32),
        pltpu.VMEM((C1,), jnp.int32),
        pltpu.SemaphoreType.DMA,
        pltpu.VMEM((CR_S,), jnp.int32),
        pltpu.VMEM((CR_S, H), jnp.float32),
        pltpu.VMEM((CR_S,), jnp.int32),
        pltpu.VMEM((C1,), jnp.int32),
        pltpu.VMEM((C1, H), jnp.float32),
        pltpu.VMEM_SHARED((AROWS, H), jnp.float32),
    ],
)
def _seg(mh_hbm, dst_hbm, agg_hbm,
         di0, mb0, dr0, sm0, di1, mb1, dr1, sm1,
         di2, mb2, dr2, zi, zb, acc):
    c = lax.axis_index("c")
    s = lax.axis_index("s")
    iota = lax.iota(jnp.int32, 16)
    slots = ((di0, mb0, dr0, sm0), (di1, mb1, dr1, sm1))

    for p in (0, 1):
        @pl.loop(0, C1)
        def _(r):
            zb[r, pl.ds(0, 16)] = jnp.zeros((16,), jnp.float32)
            zb[r, pl.ds(16, 16)] = jnp.zeros((16,), jnp.float32)

        @pl.loop(0, SRB // C1)
        def _(j):
            base = s * SRB + j * C1
            for k in range(C1 // 16):
                zi[pl.ds(k * 16, 16)] = base + k * 16 + iota
            pltpu.sync_copy(zb, acc.at[zi])

        plsc.subcore_barrier()

        base0 = s * ES

        def remap(di_, dr_, n):
            @pl.loop(0, n // 16)
            def _(g):
                sl = pl.ds(g * 16, 16)
                v = di_[sl]
                if p == 0:
                    dr_[sl] = jnp.minimum(jnp.maximum(v, 0), NH)
                else:
                    dr_[sl] = jnp.maximum(jnp.minimum(2 * NH - 1 - v, NH), 0)

        def issue(k, slot):
            di_, mb_, dr_, sm_ = slot
            b = base0 + k * C1
            pltpu.sync_copy(dst_hbm.at[pl.ds(b, C1)], di_)
            pltpu.async_copy(mh_hbm.at[pl.ds(c * E + b, C1)], mb_, sm_)
            remap(di_, dr_, C1)

        def finish(k, slot):
            di_, mb_, dr_, sm_ = slot
            b = base0 + k * C1
            pltpu.make_async_copy(mh_hbm.at[pl.ds(c * E + b, C1)], mb_, sm_).wait()
            pltpu.sync_copy(mb_, acc.at[dr_], add=True)

        issue(0, slots[0])

        @pl.loop(0, (NCH_S - 2) // 2)
        def _(j):
            k0 = 2 * j
            issue(k0 + 1, slots[1])
            finish(k0, slots[0])
            issue(k0 + 2, slots[0])
            finish(k0 + 1, slots[1])

        finish(NCH_S - 2, slots[0])
        issue(NCH_S - 1, slots[1])
        finish(NCH_S - 1, slots[1])

        b = base0 + NCH_S * C1
        pltpu.sync_copy(dst_hbm.at[pl.ds(b, CR_S)], di2)
        pltpu.sync_copy(mh_hbm.at[pl.ds(c * E + b, CR_S)], mb2)
        @pl.loop(0, CR_S // 16)
        def _(g):
            sl = pl.ds(g * 16, 16)
            v = di2[sl]
            if p == 0:
                dr2[sl] = jnp.minimum(jnp.maximum(v, 0), NH)
            else:
                dr2[sl] = jnp.maximum(jnp.minimum(2 * NH - 1 - v, NH), 0)
        pltpu.sync_copy(mb2, acc.at[dr2], add=True)

        plsc.subcore_barrier()

        @pl.loop(0, SRB // C1)
        def _(j):
            base = s * SRB + j * C1
            for k in range(C1 // 16):
                zi[pl.ds(k * 16, 16)] = base + k * 16 + iota
            pltpu.sync_copy(acc.at[zi], zb)
            pltpu.sync_copy(zb, agg_hbm.at[c, p, pl.ds(base, C1)])


def _unseg(agg):
    return jnp.concatenate([agg[:, 0, :NH], agg[:, 1, :NH][:, ::-1]], axis=1)


def kernel(pos, edge_attr, knn_label, edge_index, W_node, b_node, W_dist, b_dist, W_knn,
           Wn0, bn0, Wn1, bn1, We0, be0, We1, be1, We2, be2, Wm1, bm1, a_prelu, Wm2, bm2):
    src = edge_index[0]
    dst = edge_index[1]

    x0p = _node0(pos, W_node, b_node.reshape(1, U))
    _DBG = False
    if _DBG:
        xs0 = jnp.take(x0p[:, :U], src, axis=0)
    else:
        xs0 = _gsum_x(x0p, src, dst)
    e0h, m0h = _edge_init(edge_attr.reshape(E, 1), knn_label, xs0,
                          W_dist, b_dist.reshape(1, H), W_knn)

    agg0h = _unseg(_seg(m0h.reshape(2 * E, H), dst))
    x1p = _node_upd(x0p, agg0h, Wn0, bn0.reshape(1, U))

    if _DBG:
        xs1 = jnp.take(x1p[:, :U], src, axis=0)
        sum1 = xs1 + jnp.take(x1p[:, :U], dst, axis=0)
    else:
        xs1, sum1 = _gsum_xs(x1p, src, dst)
    e1h, m1h = _edge_upd(e0h, sum1, xs1, We0, be0.reshape(1, U))

    agg1h = _unseg(_seg(m1h.reshape(2 * E, H), dst))
    x2p = _node_upd(x1p, agg1h, Wn1, bn1.reshape(1, U))

    if _DBG:
        sum2 = jnp.take(x2p[:, :U], src, axis=0) + jnp.take(x2p[:, :U], dst, axis=0)
    else:
        sum2 = _gsum_s(x2p, src, dst)
    out = _tail(e1h, sum2, We1, be1.reshape(1, U), We2, be2.reshape(1, U),
                Wm1, bm1.reshape(1, H), a_prelu.reshape(1, 1), Wm2, bm2.reshape(1, 1))
    return out


# speedup vs baseline: 1.9123x; 1.0680x over previous
import functools

import jax
import jax.numpy as jnp
from jax import lax
from jax.experimental import pallas as pl
from jax.experimental.pallas import tpu as pltpu
from jax.experimental.pallas import tpu_sc as plsc

N = 50000
E = 800000
U = 64
H = 32
P = 128

NC = 2
NS = 16
NW = NC * NS

EB = 4000
NB = 5000

_mesh = plsc.VectorSubcoreMesh(core_axis_name="c", subcore_axis_name="s")


def _node0_body(pos_ref, w_ref, b_ref, xp_ref):
    p = pos_ref[...]
    w = w_ref[...]
    x = p[:, 0:1] * w[0:1, :] + p[:, 1:2] * w[1:2, :] + b_ref[...]
    xp_ref[...] = jnp.concatenate([x, jnp.zeros_like(x)], axis=1)


def _node0(pos, w, b):
    return pl.pallas_call(
        _node0_body,
        grid=(N // NB,),
        in_specs=[
            pl.BlockSpec((NB, 2), lambda i: (i, 0)),
            pl.BlockSpec((2, U), lambda i: (0, 0)),
            pl.BlockSpec((1, U), lambda i: (0, 0)),
        ],
        out_specs=pl.BlockSpec((NB, P), lambda i: (i, 0)),
        out_shape=jax.ShapeDtypeStruct((N, P), jnp.float32),
    )(pos, w, b)


def _node_upd_body(xp_ref, aggh_ref, w_ref, b_ref, xn_ref):
    x = xp_ref[:, :U]
    agg = jnp.concatenate([aggh_ref[0], aggh_ref[1]], axis=1)
    t = jnp.dot(x + agg, w_ref[...], preferred_element_type=jnp.float32)
    xn = x + jnp.maximum(t + b_ref[...], 0.0)
    xn_ref[...] = jnp.concatenate([xn, jnp.zeros_like(xn)], axis=1)


def _node_upd(xp, aggh, w, b):
    return pl.pallas_call(
        _node_upd_body,
        grid=(N // NB,),
        in_specs=[
            pl.BlockSpec((NB, P), lambda i: (i, 0)),
            pl.BlockSpec((2, NB, H), lambda i: (0, i, 0)),
            pl.BlockSpec((U, U), lambda i: (0, 0)),
            pl.BlockSpec((1, U), lambda i: (0, 0)),
        ],
        out_specs=pl.BlockSpec((NB, P), lambda i: (i, 0)),
        out_shape=jax.ShapeDtypeStruct((N, P), jnp.float32),
    )(xp, aggh, w, b)


def _edge_init_body(ea_ref, knn_ref, xs_ref, wd_ref, bd_ref, wk_ref,
                    e_ref, mh_ref):
    lo = ea_ref[...] * wd_ref[...] + bd_ref[...]
    hi = jnp.dot(knn_ref[...], wk_ref[...], preferred_element_type=jnp.float32)
    xs = xs_ref[...]
    e_ref[...] = jnp.concatenate([lo, hi], axis=1)
    mh_ref[0] = xs[:, :H] * lo
    mh_ref[1] = xs[:, H:] * hi


def _edge_init(ea, knn, xs, wd, bd, wk):
    return pl.pallas_call(
        _edge_init_body,
        grid=(E // EB,),
        in_specs=[
            pl.BlockSpec((EB, 1), lambda i: (i, 0)),
            pl.BlockSpec((EB, 16), lambda i: (i, 0)),
            pl.BlockSpec((EB, U), lambda i: (i, 0)),
            pl.BlockSpec((1, H), lambda i: (0, 0)),
            pl.BlockSpec((1, H), lambda i: (0, 0)),
            pl.BlockSpec((16, H), lambda i: (0, 0)),
        ],
        out_specs=[
            pl.BlockSpec((EB, U), lambda i: (i, 0)),
            pl.BlockSpec((2, EB, H), lambda i: (0, i, 0)),
        ],
        out_shape=[
            jax.ShapeDtypeStruct((E, U), jnp.float32),
            jax.ShapeDtypeStruct((2, E, H), jnp.float32),
        ],
    )(ea, knn, xs, wd, bd, wk)


def _edge_upd_body(e_ref, sum_ref, xs_ref, w_ref, b_ref, en_ref, mh_ref):
    e = e_ref[...]
    s = sum_ref[...] + e
    t = jnp.dot(s, w_ref[...], preferred_element_type=jnp.float32)
    en = e + jnp.maximum(t + b_ref[...], 0.0)
    msg = xs_ref[...] * en
    en_ref[...] = en
    mh_ref[0] = msg[:, :H]
    mh_ref[1] = msg[:, H:]


def _edge_upd(eh, s, xs, w, b):
    return pl.pallas_call(
        _edge_upd_body,
        grid=(E // EB,),
        in_specs=[
            pl.BlockSpec((EB, U), lambda i: (i, 0)),
            pl.BlockSpec((EB, U), lambda i: (i, 0)),
            pl.BlockSpec((EB, U), lambda i: (i, 0)),
            pl.BlockSpec((U, U), lambda i: (0, 0)),
            pl.BlockSpec((1, U), lambda i: (0, 0)),
        ],
        out_specs=[
            pl.BlockSpec((EB, U), lambda i: (i, 0)),
            pl.BlockSpec((2, EB, H), lambda i: (0, i, 0)),
        ],
        out_shape=[
            jax.ShapeDtypeStruct((E, U), jnp.float32),
            jax.ShapeDtypeStruct((2, E, H), jnp.float32),
        ],
    )(eh, s, xs, w, b)


def _tail_body(e_ref, sum_ref, w1_ref, b1_ref, w2_ref, b2_ref,
               wm1_ref, bm1_ref, a_ref, wm2_ref, bm2_ref, o_ref):
    e1 = e_ref[...]
    s2 = sum_ref[...]
    t1 = jnp.dot(s2 + e1, w1_ref[...], preferred_element_type=jnp.float32)
    e2 = e1 + jnp.maximum(t1 + b1_ref[...], 0.0)
    t2 = jnp.dot(s2 + e2, w2_ref[...], preferred_element_type=jnp.float32)
    e3 = e2 + jnp.maximum(t2 + b2_ref[...], 0.0)
    h = jnp.dot(e3, wm1_ref[...], preferred_element_type=jnp.float32) + bm1_ref[...]
    a = a_ref[0, 0]
    h = jnp.where(h >= 0, h, a * h)
    o_ref[...] = jnp.dot(h, wm2_ref[...], preferred_element_type=jnp.float32) + bm2_ref[...]


def _tail(eh, s, w1, b1, w2, b2, wm1, bm1, a, wm2, bm2):
    return pl.pallas_call(
        _tail_body,
        grid=(E // EB,),
        in_specs=[
            pl.BlockSpec((EB, U), lambda i: (i, 0)),
            pl.BlockSpec((EB, U), lambda i: (i, 0)),
            pl.BlockSpec((U, U), lambda i: (0, 0)),
            pl.BlockSpec((1, U), lambda i: (0, 0)),
            pl.BlockSpec((U, U), lambda i: (0, 0)),
            pl.BlockSpec((1, U), lambda i: (0, 0)),
            pl.BlockSpec((U, H), lambda i: (0, 0)),
            pl.BlockSpec((1, H), lambda i: (0, 0)),
            pl.BlockSpec((1, 1), lambda i: (0, 0)),
            pl.BlockSpec((H, 1), lambda i: (0, 0)),
            pl.BlockSpec((1, 1), lambda i: (0, 0)),
        ],
        out_specs=pl.BlockSpec((EB, 1), lambda i: (i, 0)),
        out_shape=jax.ShapeDtypeStruct((E, 1), jnp.float32),
    )(eh, s, w1, b1, w2, b2, wm1, bm1, a, wm2, bm2)


C1 = 128

EW = E // NW
NCH_W = EW // C1
CR_W = EW - NCH_W * C1

ES = E // NS
NCH_S = ES // C1
CR_S = ES - NCH_S * C1

_SEG_ON = True
NH = 25000
SRB = 1664
AROWS = 16 * SRB


def _make_gsum(want_xs, want_sum):
    out_type = []
    if want_xs:
        out_type.append(jax.ShapeDtypeStruct((E, U), jnp.float32))
    if want_sum:
        out_type.append(jax.ShapeDtypeStruct((E, U), jnp.float32))
    n_out = len(out_type)
    if n_out == 1:
        out_type = out_type[0]

    scratch = []
    for _ in range(2):
        scratch += [
            pltpu.VMEM((C1,), jnp.int32),
            pltpu.VMEM((C1,), jnp.int32),
            pltpu.VMEM((C1, P), jnp.float32),
            pltpu.VMEM((C1, P), jnp.float32),
            pltpu.SemaphoreType.DMA,
            pltpu.SemaphoreType.DMA,
        ]
    scratch += [
        pltpu.VMEM((C1, U), jnp.float32),
        pltpu.VMEM((CR_W,), jnp.int32),
        pltpu.VMEM((CR_W,), jnp.int32),
        pltpu.VMEM((CR_W, P), jnp.float32),
        pltpu.VMEM((CR_W, U), jnp.float32),
    ]

    @functools.partial(pl.kernel, out_type=out_type, mesh=_mesh,
                       scratch_types=scratch)
    def gsum(xp_hbm, src_hbm, dst_hbm, *rest):
        outs = list(rest[:n_out])
        (si0, di0, bs0, bd0, ss0, sd0,
         si1, di1, bs1, bd1, ss1, sd1,
         stg, si2, di2, buf2, stg2) = rest[n_out:]
        xs_hbm = outs.pop(0) if want_xs else None
        sum_hbm = outs.pop(0) if want_sum else None
        slots = ((si0, di0, bs0, bd0, ss0, sd0),
                 (si1, di1, bs1, bd1, ss1, sd1))

        c = lax.axis_index("c")
        s = lax.axis_index("s")
        wid = s * NC + c
        base0 = wid * EW

        def issue(k, slot):
            si_, di_, bs_, bd_, ss_, sd_ = slot
            b = base0 + k * C1
            pltpu.sync_copy(src_hbm.at[pl.ds(b, C1)], si_)
            pltpu.async_copy(xp_hbm.at[si_], bs_, ss_)
            if want_sum:
                pltpu.sync_copy(dst_hbm.at[pl.ds(b, C1)], di_)
                pltpu.async_copy(xp_hbm.at[di_], bd_, sd_)

        def finish(k, slot):
            si_, di_, bs_, bd_, ss_, sd_ = slot
            b = base0 + k * C1
            pltpu.make_async_copy(xp_hbm.at[si_], bs_, ss_).wait()
            if want_sum:
                pltpu.make_async_copy(xp_hbm.at[di_], bd_, sd_).wait()
            if want_xs:
                @pl.loop(0, C1)
                def _(r):
                    for cc in range(U // 16):
                        sl = pl.ds(cc * 16, 16)
                        stg[r, sl] = bs_[r, sl]
                pltpu.sync_copy(stg, xs_hbm.at[pl.ds(b, C1)])
            if want_sum:
                @pl.loop(0, C1)
                def _(r):
                    for cc in range(U // 16):
                        sl = pl.ds(cc * 16, 16)
                        stg[r, sl] = bs_[r, sl] + bd_[r, sl]
                pltpu.sync_copy(stg, sum_hbm.at[pl.ds(b, C1)])

        issue(0, slots[0])

        @pl.loop(0, (NCH_W - 1) // 2)
        def _(j):
            k0 = 2 * j
            issue(k0 + 1, slots[1])
            finish(k0, slots[0])
            issue(k0 + 2, slots[0])
            finish(k0 + 1, slots[1])

        finish(NCH_W - 1, slots[0])

        b = base0 + NCH_W * C1
        pltpu.sync_copy(src_hbm.at[pl.ds(b, CR_W)], si2)
        pltpu.sync_copy(xp_hbm.at[si2], buf2)
        if want_xs:
            @pl.loop(0, CR_W)
            def _(r):
                for cc in range(U // 16):
                    sl = pl.ds(cc * 16, 16)
                    stg2[r, sl] = buf2[r, sl]
            pltpu.sync_copy(stg2, xs_hbm.at[pl.ds(b, CR_W)])
        if want_sum:
            pltpu.sync_copy(dst_hbm.at[pl.ds(b, CR_W)], di2)
            pltpu.sync_copy(xp_hbm.at[di2], buf2, add=True)
            @pl.loop(0, CR_W)
            def _(r):
                for cc in range(U // 16):
                    sl = pl.ds(cc * 16, 16)
                    stg2[r, sl] = buf2[r, sl]
            pltpu.sync_copy(stg2, sum_hbm.at[pl.ds(b, CR_W)])

    return gsum


_gsum_x = _make_gsum(True, False)
_gsum_xs = _make_gsum(True, True)
_gsum_s = _make_gsum(False, True)


@functools.partial(
    pl.kernel,
    out_type=jax.ShapeDtypeStruct((2, 2, AROWS, H), jnp.float32),
    mesh=_mesh,
    scratch_types=[
        pltpu.VMEM((C1,), jnp.int32),
        pltpu.VMEM((C1, H), jnp.float32),
        pltpu.VMEM((C1,), jnp.int32),
        pltpu.SemaphoreType.DMA,
        pltpu.VMEM((C1,), jnp.int32),
        pltpu.VMEM((C1, H), jnp.float32),
        pltpu.VMEM((C1,), jnp.int32),
        pltpu.SemaphoreType.DMA,
        pltpu.VMEM((CR_S,), jnp.int32),
        pltpu.VMEM((CR_S, H), jnp.float32),
        pltpu.VMEM((CR_S,), jnp.int32),
        pltpu.VMEM((C1,), jnp.int32),
        pltpu.VMEM((C1, H), jnp.float32),
        pltpu.VMEM_SHARED((AROWS, H), jnp.float32),
    ],
)
def _seg(mh_hbm, dst_hbm, agg_hbm,
         di0, mb0, dr0, sm0, di1, mb1, dr1, sm1,
         di2, mb2, dr2, zi, zb, acc):
    c = lax.axis_index("c")
    s = lax.axis_index("s")
    iota = lax.iota(jnp.int32, 16)
    slots = ((di0, mb0, dr0, sm0), (di1, mb1, dr1, sm1))

    for p in (0, 1):
        @pl.loop(0, C1)
        def _(r):
            zb[r, pl.ds(0, 16)] = jnp.zeros((16,), jnp.float32)
            zb[r, pl.ds(16, 16)] = jnp.zeros((16,), jnp.float32)

        @pl.loop(0, SRB // C1)
        def _(j):
            base = s * SRB + j * C1
            for k in range(C1 // 16):
                zi[pl.ds(k * 16, 16)] = base + k * 16 + iota
            pltpu.sync_copy(zb, acc.at[zi])

        plsc.subcore_barrier()

        base0 = s * ES

        def remap(di_, dr_, n):
            @pl.loop(0, n // 16)
            def _(g):
                sl = pl.ds(g * 16, 16)
                v = di_[sl]
                if p == 0:
                    dr_[sl] = jnp.minimum(jnp.maximum(v, 0), NH)
                else:
                    dr_[sl] = jnp.maximum(jnp.minimum(2 * NH - 1 - v, NH), 0)

        def issue(k, slot):
            di_, mb_, dr_, sm_ = slot
            b = base0 + k * C1
            pltpu.sync_copy(dst_hbm.at[pl.ds(b, C1)], di_)
            pltpu.async_copy(mh_hbm.at[pl.ds(c * E + b, C1)], mb_, sm_)
            remap(di_, dr_, C1)

        def finish(k, slot):
            di_, mb_, dr_, sm_ = slot
            b = base0 + k * C1
            pltpu.make_async_copy(mh_hbm.at[pl.ds(c * E + b, C1)], mb_, sm_).wait()
            pltpu.sync_copy(mb_, acc.at[dr_], add=True)

        issue(0, slots[0])

        @pl.loop(0, (NCH_S - 2) // 2)
        def _(j):
            k0 = 2 * j
            issue(k0 + 1, slots[1])
            finish(k0, slots[0])
            issue(k0 + 2, slots[0])
            finish(k0 + 1, slots[1])

        finish(NCH_S - 2, slots[0])
        issue(NCH_S - 1, slots[1])
        finish(NCH_S - 1, slots[1])

        b = base0 + NCH_S * C1
        pltpu.sync_copy(dst_hbm.at[pl.ds(b, CR_S)], di2)
        pltpu.sync_copy(mh_hbm.at[pl.ds(c * E + b, CR_S)], mb2)
        @pl.loop(0, CR_S // 16)
        def _(g):
            sl = pl.ds(g * 16, 16)
            v = di2[sl]
            if p == 0:
                dr2[sl] = jnp.minimum(jnp.maximum(v, 0), NH)
            else:
                dr2[sl] = jnp.maximum(jnp.minimum(2 * NH - 1 - v, NH), 0)
        pltpu.sync_copy(mb2, acc.at[dr2], add=True)

        plsc.subcore_barrier()

        @pl.loop(0, SRB // C1)
        def _(j):
            base = s * SRB + j * C1
            for k in range(C1 // 16):
                zi[pl.ds(k * 16, 16)] = base + k * 16 + iota
            pltpu.sync_copy(acc.at[zi], zb)
            pltpu.sync_copy(zb, agg_hbm.at[c, p, pl.ds(base, C1)])


def _unseg(agg):
    return jnp.concatenate([agg[:, 0, :NH], agg[:, 1, :NH][:, ::-1]], axis=1)


def kernel(pos, edge_attr, knn_label, edge_index, W_node, b_node, W_dist, b_dist, W_knn,
           Wn0, bn0, Wn1, bn1, We0, be0, We1, be1, We2, be2, Wm1, bm1, a_prelu, Wm2, bm2):
    src = edge_index[0]
    dst = edge_index[1]

    x0p = _node0(pos, W_node, b_node.reshape(1, U))
    _DBG = False
    if _DBG:
        xs0 = jnp.take(x0p[:, :U], src, axis=0)
    else:
        xs0 = _gsum_x(x0p, src, dst)
    e0h, m0h = _edge_init(edge_attr.reshape(E, 1), knn_label, xs0,
                          W_dist, b_dist.reshape(1, H), W_knn)

    agg0h = _unseg(_seg(m0h.reshape(2 * E, H), dst))
    x1p = _node_upd(x0p, agg0h, Wn0, bn0.reshape(1, U))

    if _DBG:
        xs1 = jnp.take(x1p[:, :U], src, axis=0)
        sum1 = xs1 + jnp.take(x1p[:, :U], dst, axis=0)
    else:
        xs1, sum1 = _gsum_xs(x1p, src, dst)
    e1h, m1h = _edge_upd(e0h, sum1, xs1, We0, be0.reshape(1, U))

    agg1h = _unseg(_seg(m1h.reshape(2 * E, H), dst))
    x2p = _node_upd(x1p, agg1h, Wn1, bn1.reshape(1, U))

    if _DBG:
        sum2 = jnp.take(x2p[:, :U], src, axis=0) + jnp.take(x2p[:, :U], dst, axis=0)
    else:
        sum2 = _gsum_s(x2p, src, dst)
    out = _tail(e1h, sum2, We1, be1.reshape(1, U), We2, be2.reshape(1, U),
                Wm1, bm1.reshape(1, H), a_prelu.reshape(1, 1), Wm2, bm2.reshape(1, 1))
    return out
